# baseline (speedup 1.0000x reference)
import functools

import jax
import jax.numpy as jnp
from jax.experimental import pallas as pl
from jax.experimental.pallas import tpu as pltpu

B, K, D, H, A = 1024, 10, 128, 32, 16
SIGMA = 0.5
NE = B * (K - 1)
NT = B * K - NE
NB = 2
GR = NE // NB
TR = NT // NB
EPS = 1e-5


def _body(xg_ref, xt_ref, nxg_ref, nxt_ref, avg_ref, avt_ref,
          wsp_ref, weap_ref, wt_ref, eb14_ref, w2c_ref, b2c_ref, mavg_ref,
          eg14_ref, ebt14_ref, w3s_ref, eb38_ref, nf_ref, na_ref, ng_ref,
          nb1_ref, nw2c_ref, nb2c_ref, avgh_ref, ng1_ref, nbt1_ref, nw3_ref,
          nb3_ref, out_ref):
    f32 = jnp.float32
    dot = functools.partial(jnp.dot, preferred_element_type=f32)

    X = jnp.concatenate([xg_ref[...], xt_ref[...]], axis=0)
    av = jnp.concatenate([avg_ref[...], avt_ref[...]], axis=0)

    wsp = wsp_ref[...]
    weap = weap_ref[...]
    wt = wt_ref[...]
    eb14 = eb14_ref[...]
    w2c = w2c_ref[...]
    b2c = b2c_ref[...]
    mavg = mavg_ref[...]
    eg14 = eg14_ref[...]
    ebt14 = ebt14_ref[...]
    w3s = w3s_ref[...]
    eb38 = eb38_ref[...]
    nf = nf_ref[...]
    na_w = na_ref[...]
    ng_w = ng_ref[...]
    nb1 = nb1_ref[...]
    nw2c = nw2c_ref[...]
    nb2c = nb2c_ref[...]
    avgh = avgh_ref[...]
    ng1 = ng1_ref[...]
    nbt1 = nbt1_ref[...]
    nw3 = nw3_ref[...]
    nb3 = nb3_ref[...]

    def edge_agg(Spack, T):
        GPB = GR // 9
        rolls = [
            jnp.concatenate([T[d * GPB:], T[:d * GPB]], axis=0)
            for d in range(1, 9)
        ]
        e1 = jax.nn.relu(
            Spack + jnp.concatenate(rolls[:4], axis=1) + eb14)
        e2 = jax.nn.relu(
            Spack + jnp.concatenate(rolls[4:], axis=1) + eb14)
        h = jnp.concatenate([e1, e2], axis=0)
        c = dot(h, w2c) + b2c
        v = dot(c * c, mavg)
        h3 = jax.nn.relu(c * jax.lax.rsqrt(v + EPS) * eg14 + ebt14)
        hs = h3[:GR] + h3[GR:]
        return dot(hs, w3s) + eb38

    def node_stage(Xc, agg_all, av_term):
        h = dot(Xc, nf) + dot(agg_all, ng_w) + nb1 + av_term
        h = jax.nn.relu(h)
        c = dot(h, nw2c) + nb2c
        v = dot(c * c, avgh)
        h3 = jax.nn.relu(c * jax.lax.rsqrt(v + EPS) * ng1 + nbt1)
        return dot(h3, nw3) + nb3

    zeros_tail = jnp.zeros((TR, H), f32)

    Xg = X[:GR]
    agg = edge_agg(dot(Xg, wsp), dot(Xg, wt))
    na = node_stage(X, jnp.concatenate([agg, zeros_tail], axis=0),
                    dot(av, na_w))
    X = X + na

    for _ in range(K - 1):
        Xg = X[:GR]
        agg = edge_agg(dot(Xg, wsp) + dot(na[:GR], weap), dot(Xg, wt))
        na = node_stage(X, jnp.concatenate([agg, zeros_tail], axis=0), 0.0)
        X = X + na

    nx = jnp.concatenate([nxg_ref[...], nxt_ref[...]], axis=0)
    d2 = (X - nx) ** 2
    s = jnp.sum(jnp.sum(d2, axis=1, keepdims=True), axis=0, keepdims=True)
    scale = (0.5 / (SIGMA ** 2)) / (B * K)
    out_ref[...] = jnp.zeros((1, 8, 128), f32) + s * scale


def kernel(state, action, next_state, ew1, eb1, ew2, eb2, eg1, ebt1, ew3,
           eb3, nw1, nb1, nw2, nb2, ng1, nbt1, nw3, nb3):
    f32 = jnp.float32
    flat = state.reshape(B * K, D).astype(f32)
    nflat = next_state.reshape(B * K, D).astype(f32)
    av = jnp.repeat(jax.nn.one_hot(action, A, dtype=f32), K, axis=0)

    GPB = GR // 9
    sm = lambda a: a[:NE].reshape(NB, GPB, 9, -1).transpose(0, 2, 1, 3) \
                         .reshape(NE, -1)

    row2 = lambda v: v.reshape(1, -1).astype(f32)
    tile4 = lambda v: jnp.concatenate([v.astype(f32)] * 4, axis=-1)
    bdiag4 = lambda m: jax.scipy.linalg.block_diag(*([m.astype(f32)] * 4))

    ws, wea, wt = ew1[:D], ew1[D:2 * D], ew1[2 * D:]
    w2p = bdiag4(ew2)
    avgh = jnp.full((H, H), 1.0 / H, f32)
    mavg = bdiag4(avgh)
    w2c = w2p - w2p @ mavg
    eb24 = tile4(row2(eb2))
    b2c = eb24 - eb24 @ mavg
    nw2c = nw2.astype(f32) - nw2.astype(f32) @ avgh
    nb2c = row2(nb2) - row2(nb2) @ avgh
    weights = [
        tile4(ws), tile4(wea), wt.astype(f32),
        tile4(row2(eb1)), w2c, b2c, mavg,
        tile4(row2(eg1)), tile4(row2(ebt1)),
        jnp.concatenate([ew3.astype(f32)] * 4, axis=0), 8.0 * row2(eb3),
        nw1[:D].astype(f32), nw1[D:D + A].astype(f32), nw1[D + A:].astype(f32),
        row2(nb1), nw2c, nb2c, avgh, row2(ng1), row2(nbt1),
        nw3.astype(f32), row2(nb3),
    ]

    full = lambda a: pl.BlockSpec(a.shape, lambda i: (0, 0))
    in_specs = [
        pl.BlockSpec((GR, D), lambda i: (i, 0)),
        pl.BlockSpec((TR, D), lambda i: (i, 0)),
        pl.BlockSpec((GR, D), lambda i: (i, 0)),
        pl.BlockSpec((TR, D), lambda i: (i, 0)),
        pl.BlockSpec((GR, A), lambda i: (i, 0)),
        pl.BlockSpec((TR, A), lambda i: (i, 0)),
    ] + [full(w) for w in weights]

    out = pl.pallas_call(
        _body,
        grid=(NB,),
        in_specs=in_specs,
        out_specs=pl.BlockSpec((1, 8, 128), lambda i: (i, 0, 0)),
        out_shape=jax.ShapeDtypeStruct((NB, 8, 128), f32),
        compiler_params=pltpu.CompilerParams(
            dimension_semantics=("parallel",)),
    )(sm(flat), flat[NE:], sm(nflat), nflat[NE:], sm(av), av[NE:],
      *weights)
    return jnp.sum(out[:, 0, 0])


# speedup vs baseline: 1.1753x; 1.1753x over previous
import functools

import jax
import jax.numpy as jnp
from jax.experimental import pallas as pl
from jax.experimental.pallas import tpu as pltpu

B, K, D, H, A = 1024, 10, 128, 32, 16
SIGMA = 0.5
N = B * K
NE = B * (K - 1)
NT = N - NE
NB = 4
GR = NE // NB
TR = NT // NB
GPB = GR // 9
EPS = 1e-5


def _body(xs_ref, xt_ref, nxs_ref, nxt_ref, as_ref, at_ref, w32_ref,
          w128_ref, out_ref):
    f32 = jnp.float32
    dot = functools.partial(jnp.dot, preferred_element_type=f32)
    cat = jnp.concatenate

    w32 = w32_ref[...]
    ws, wea, wt = w32[0:D], w32[D:2 * D], w32[2 * D:3 * D]
    nf = w32[384:512]
    na_w = w32[512:528]
    ng_w = w32[528:560]
    nw2 = w32[560:592]
    ew2 = w32[592:624]
    ew3 = w32[624:656]
    eb1 = w32[656:657]
    eb2 = w32[657:658]
    eg1 = w32[658:659]
    ebt1 = w32[659:660]
    eb3 = w32[660:661]
    nb1 = w32[661:662]
    nb2 = w32[662:663]
    ng1 = w32[663:664]
    nbt1 = w32[664:665]
    nw3 = w128_ref[0:H]
    nb3 = w128_ref[H:H + 1]

    tile4 = lambda a: cat([a] * 4, axis=1)
    avgh = jnp.full((H, H), 1.0 / H, f32)
    z = jnp.zeros((H, H), f32)

    def bdiag4(m):
        return cat([cat([m if j == i else z for j in range(4)], axis=1)
                    for i in range(4)], axis=0)

    wsp = tile4(ws)
    weap = tile4(wea)
    w2c = bdiag4(ew2 - dot(ew2, avgh))
    mavg = bdiag4(avgh)
    b2c = tile4(eb2 - dot(eb2, avgh))
    eb14 = tile4(eb1)
    eg14 = tile4(eg1)
    ebt14 = tile4(ebt1)
    w3s = cat([ew3] * 4, axis=0)
    eb38 = 8.0 * eb3
    nw2c = nw2 - dot(nw2, avgh)
    nb2c = nb2 - dot(nb2, avgh)

    X = cat([xs_ref[...], xt_ref[...]], axis=0)
    act = cat([as_ref[...], at_ref[...]], axis=0)
    av = (jax.lax.broadcasted_iota(jnp.int32, (GR + TR, A), 1)
          == act).astype(f32)

    def edge_agg(Spack, T):
        rolls = [cat([T[d * GPB:], T[:d * GPB]], axis=0)
                 for d in range(1, 9)]
        e1 = jax.nn.relu(Spack + cat(rolls[:4], axis=1))
        e2 = jax.nn.relu(Spack + cat(rolls[4:], axis=1))
        h = cat([e1, e2], axis=0)
        c = dot(h, w2c) + b2c
        v = dot(c * c, mavg)
        h3 = jax.nn.relu(c * jax.lax.rsqrt(v + EPS) * eg14 + ebt14)
        hs = h3[:GR] + h3[GR:]
        return dot(hs, w3s) + eb38

    def node_stage(Xc, agg_all, av_term):
        h = dot(Xc, nf) + dot(agg_all, ng_w) + nb1 + av_term
        h = jax.nn.relu(h)
        c = dot(h, nw2c) + nb2c
        v = dot(c * c, avgh)
        h3 = jax.nn.relu(c * jax.lax.rsqrt(v + EPS) * ng1 + nbt1)
        return dot(h3, nw3) + nb3

    zeros_tail = jnp.zeros((TR, H), f32)

    Xg = X[:GR]
    agg = edge_agg(dot(Xg, wsp) + eb14, dot(Xg, wt))
    na = node_stage(X, cat([agg, zeros_tail], axis=0), dot(av, na_w))
    X = X + na

    for _ in range(K - 1):
        Xg = X[:GR]
        agg = edge_agg(dot(Xg, wsp) + dot(na[:GR], weap) + eb14, dot(Xg, wt))
        na = node_stage(X, cat([agg, zeros_tail], axis=0), 0.0)
        X = X + na

    nx = cat([nxs_ref[...], nxt_ref[...]], axis=0)
    d2 = (X - nx) ** 2
    s = jnp.sum(jnp.sum(d2, axis=1, keepdims=True), axis=0, keepdims=True)
    scale = (0.5 / (SIGMA ** 2)) / (B * K)
    out_ref[...] = jnp.zeros((1, 8, 128), f32) + s * scale


def kernel(state, action, next_state, ew1, eb1, ew2, eb2, eg1, ebt1, ew3,
           eb3, nw1, nb1, nw2, nb2, ng1, nbt1, nw3, nb3):
    f32 = jnp.float32
    flat = state.reshape(N, D).astype(f32)
    nflat = next_state.reshape(N, D).astype(f32)
    act_rep = jnp.repeat(action.astype(jnp.int32), K).reshape(N, 1)

    sm = lambda a: a[:NE].reshape(NB, GPB, 9, -1).transpose(0, 2, 1, 3) \
                         .reshape(NE, -1)

    row2 = lambda v: v.reshape(1, -1).astype(f32)
    w32 = jnp.concatenate([
        ew1.astype(f32), nw1.astype(f32), nw2.astype(f32), ew2.astype(f32),
        ew3.astype(f32), row2(eb1), row2(eb2), row2(eg1), row2(ebt1),
        row2(eb3), row2(nb1), row2(nb2), row2(ng1), row2(nbt1),
    ], axis=0)
    w128 = jnp.concatenate([nw3.astype(f32), row2(nb3)], axis=0)

    TOFF = NE // TR
    in_specs = [
        pl.BlockSpec((GR, D), lambda i: (i, 0)),
        pl.BlockSpec((TR, D), lambda i: (TOFF + i, 0)),
        pl.BlockSpec((GR, D), lambda i: (i, 0)),
        pl.BlockSpec((TR, D), lambda i: (TOFF + i, 0)),
        pl.BlockSpec((GR, 1), lambda i: (i, 0)),
        pl.BlockSpec((TR, 1), lambda i: (TOFF + i, 0)),
        pl.BlockSpec((665, 32), lambda i: (0, 0)),
        pl.BlockSpec((33, 128), lambda i: (0, 0)),
    ]

    out = pl.pallas_call(
        _body,
        grid=(NB,),
        in_specs=in_specs,
        out_specs=pl.BlockSpec((1, 8, 128), lambda i: (i, 0, 0)),
        out_shape=jax.ShapeDtypeStruct((NB, 8, 128), f32),
        compiler_params=pltpu.CompilerParams(
            dimension_semantics=("parallel",)),
    )(sm(flat), flat, sm(nflat), nflat, sm(act_rep), act_rep, w32, w128)
    return jnp.sum(out[:, 0, 0])


# speedup vs baseline: 1.2127x; 1.0318x over previous
import functools

import jax
import jax.numpy as jnp
from jax.experimental import pallas as pl
from jax.experimental.pallas import tpu as pltpu

B, K, D, H, A = 1024, 10, 128, 32, 16
SIGMA = 0.5
N = B * K
NE = B * (K - 1)
NT = N - NE
NB = 4
GR = NE // NB
TR = NT // NB
GPB = GR // 9
EPS = 1e-5


def _body(xs_ref, xt_ref, nxs_ref, nxt_ref, as_ref, at_ref, w32_ref,
          w128_ref, out_ref):
    f32 = jnp.float32
    dot = functools.partial(jnp.dot, preferred_element_type=f32)
    cat = jnp.concatenate

    w32 = w32_ref[...]
    ws, wea, wt = w32[0:D], w32[D:2 * D], w32[2 * D:3 * D]
    nf = w32[384:512]
    na_w = w32[512:528]
    ng_w = w32[528:560]
    nw2 = w32[560:592]
    ew2 = w32[592:624]
    ew3 = w32[624:656]
    eb1 = w32[656:657]
    eb2 = w32[657:658]
    eg1 = w32[658:659]
    ebt1 = w32[659:660]
    eb3 = w32[660:661]
    nb1 = w32[661:662]
    nb2 = w32[662:663]
    ng1 = w32[663:664]
    nbt1 = w32[664:665]
    nw3 = w128_ref[0:H]
    nb3 = w128_ref[H:H + 1]

    tile4 = lambda a: cat([a] * 4, axis=1)
    avgh = jnp.full((H, H), 1.0 / H, f32)
    z = jnp.zeros((H, H), f32)

    def bdiag4(m):
        return cat([cat([m if j == i else z for j in range(4)], axis=1)
                    for i in range(4)], axis=0)

    wsp = tile4(ws)
    weap = tile4(wea)
    w2c = bdiag4(ew2 - dot(ew2, avgh))
    mavg = bdiag4(avgh)
    b2c = tile4(eb2 - dot(eb2, avgh))
    eb14 = tile4(eb1)
    eg14 = tile4(eg1)
    ebt14 = tile4(ebt1)
    w3s = cat([ew3] * 4, axis=0)
    eb38 = 8.0 * eb3
    nw2c = nw2 - dot(nw2, avgh)
    nb2c = nb2 - dot(nb2, avgh)

    slotcat = lambda r: cat([r[:, s, :] for s in range(9)], axis=0)
    X = cat([slotcat(xs_ref), xt_ref[...]], axis=0)
    act = cat([slotcat(as_ref), at_ref[...]], axis=0)
    av = (jax.lax.broadcasted_iota(jnp.int32, (GR + TR, A), 1)
          == act).astype(f32)

    def edge_agg(Spack, T):
        rolls = [cat([T[d * GPB:], T[:d * GPB]], axis=0)
                 for d in range(1, 9)]
        e1 = jax.nn.relu(Spack + cat(rolls[:4], axis=1))
        e2 = jax.nn.relu(Spack + cat(rolls[4:], axis=1))
        h = cat([e1, e2], axis=0)
        c = dot(h, w2c) + b2c
        v = dot(c * c, mavg)
        h3 = jax.nn.relu(c * jax.lax.rsqrt(v + EPS) * eg14 + ebt14)
        hs = h3[:GR] + h3[GR:]
        return dot(hs, w3s) + eb38

    def node_stage(Xc, agg_all, av_term):
        h = dot(Xc, nf) + dot(agg_all, ng_w) + nb1 + av_term
        h = jax.nn.relu(h)
        c = dot(h, nw2c) + nb2c
        v = dot(c * c, avgh)
        h3 = jax.nn.relu(c * jax.lax.rsqrt(v + EPS) * ng1 + nbt1)
        return dot(h3, nw3) + nb3

    zeros_tail = jnp.zeros((TR, H), f32)

    Xg = X[:GR]
    agg = edge_agg(dot(Xg, wsp) + eb14, dot(Xg, wt))
    na = node_stage(X, cat([agg, zeros_tail], axis=0), dot(av, na_w))
    X = X + na

    for _ in range(K - 1):
        Xg = X[:GR]
        agg = edge_agg(dot(Xg, wsp) + dot(na[:GR], weap) + eb14, dot(Xg, wt))
        na = node_stage(X, cat([agg, zeros_tail], axis=0), 0.0)
        X = X + na

    nx = cat([slotcat(nxs_ref), nxt_ref[...]], axis=0)
    d2 = (X - nx) ** 2
    s = jnp.sum(jnp.sum(d2, axis=1, keepdims=True), axis=0, keepdims=True)
    scale = (0.5 / (SIGMA ** 2)) / (B * K)
    out_ref[...] = jnp.zeros((1, 8, 128), f32) + s * scale


def kernel(state, action, next_state, ew1, eb1, ew2, eb2, eg1, ebt1, ew3,
           eb3, nw1, nb1, nw2, nb2, ng1, nbt1, nw3, nb3):
    f32 = jnp.float32
    flat = state.reshape(N, D).astype(f32)
    nflat = next_state.reshape(N, D).astype(f32)
    act_rep = jnp.repeat(action.astype(jnp.int32), K).reshape(N, 1)

    g3 = lambda a: a[:NE].reshape(NB * GPB, 9, -1)

    row2 = lambda v: v.reshape(1, -1).astype(f32)
    w32 = jnp.concatenate([
        ew1.astype(f32), nw1.astype(f32), nw2.astype(f32), ew2.astype(f32),
        ew3.astype(f32), row2(eb1), row2(eb2), row2(eg1), row2(ebt1),
        row2(eb3), row2(nb1), row2(nb2), row2(ng1), row2(nbt1),
    ], axis=0)
    w128 = jnp.concatenate([nw3.astype(f32), row2(nb3)], axis=0)

    TOFF = NE // TR
    in_specs = [
        pl.BlockSpec((GPB, 9, D), lambda i: (i, 0, 0)),
        pl.BlockSpec((TR, D), lambda i: (TOFF + i, 0)),
        pl.BlockSpec((GPB, 9, D), lambda i: (i, 0, 0)),
        pl.BlockSpec((TR, D), lambda i: (TOFF + i, 0)),
        pl.BlockSpec((GPB, 9, 1), lambda i: (i, 0, 0)),
        pl.BlockSpec((TR, 1), lambda i: (TOFF + i, 0)),
        pl.BlockSpec((665, 32), lambda i: (0, 0)),
        pl.BlockSpec((33, 128), lambda i: (0, 0)),
    ]

    out = pl.pallas_call(
        _body,
        grid=(NB,),
        in_specs=in_specs,
        out_specs=pl.BlockSpec((1, 8, 128), lambda i: (i, 0, 0)),
        out_shape=jax.ShapeDtypeStruct((NB, 8, 128), f32),
        compiler_params=pltpu.CompilerParams(
            dimension_semantics=("parallel",)),
    )(g3(flat), flat, g3(nflat), nflat, g3(act_rep), act_rep, w32, w128)
    return jnp.sum(out[:, 0, 0])
